# Initial kernel scaffold; baseline (speedup 1.0000x reference)
import jax
import jax.numpy as jnp
from jax.experimental import pallas as pl


def kernel(A_rel, K_past, k_t, t):
    raise NotImplementedError("write your pallas kernel here")


# speedup vs baseline: 1.3792x; 1.3792x over previous
import functools

import jax
import jax.numpy as jnp
from jax import lax
from jax.experimental import pallas as pl
from jax.experimental.pallas import tpu as pltpu

_GAMMA = 0.5
_EPS = 1e-06
_BT = 512


def _fused_body(t_ref, arow_ref, a_ref, k_ref, kt_ref, out_ref,
                acc_ref, m_ref, degt_ref):
    c = pl.program_id(1)
    nc = pl.num_programs(1)

    @pl.when(c == 0)
    def _init():
        acc_ref[...] = jnp.zeros_like(acc_ref)
        m_ref[0] = 0.0
        degt_ref[0] = 1.0

    a = a_ref[0]
    deg = jnp.sum(a, axis=1, keepdims=True) + _EPS
    ris = lax.rsqrt(deg)
    u = arow_ref[0] * ris
    k = k_ref[0]
    acc_ref[...] += jnp.sum(k * u, axis=0, keepdims=True)
    m_ref[0] = jnp.maximum(m_ref[0], jnp.max(jnp.abs(u)))

    t = t_ref[0, 0]
    start = c * _BT
    in_chunk = jnp.logical_and(t >= start, t < start + _BT)

    @pl.when(in_chunk)
    def _corr():
        loc = t - start
        deg_t = jnp.sum(a_ref[0, pl.ds(loc, 1), :], axis=1, keepdims=True) + _EPS
        u_t = arow_ref[0, pl.ds(loc, 1), :] * lax.rsqrt(deg_t)
        krow = k_ref[0, pl.ds(loc, 1), :]
        acc_ref[...] += u_t * (kt_ref[0] - krow)
        degt_ref[0] = deg_t[0, 0]

    @pl.when(c == nc - 1)
    def _fin():
        rd = lax.rsqrt(degt_ref[0])
        mask = jnp.where(m_ref[0] * rd > 1e-9, 1.0, 0.0)
        out_ref[0] = acc_ref[...] * (rd * jnp.sqrt(_GAMMA) * mask)


def kernel(A_rel, K_past, k_t, t):
    B, T, D = K_past.shape
    nc = T // _BT
    t_i = jnp.asarray(t, jnp.int32).reshape(1, 1)
    arow3 = lax.dynamic_slice(A_rel, (0, t_i[0, 0], 0), (B, 1, T))
    arow3 = arow3.reshape(B, T, 1)

    out = pl.pallas_call(
        _fused_body,
        grid=(B, nc),
        in_specs=[
            pl.BlockSpec(memory_space=pltpu.SMEM),
            pl.BlockSpec((1, _BT, 1), lambda b, c: (b, c, 0)),
            pl.BlockSpec((1, _BT, T), lambda b, c: (b, c, 0)),
            pl.BlockSpec((1, _BT, D), lambda b, c: (b, c, 0)),
            pl.BlockSpec((1, 1, D), lambda b, c: (b, 0, 0)),
        ],
        out_specs=pl.BlockSpec((1, 1, D), lambda b, c: (b, 0, 0)),
        out_shape=jax.ShapeDtypeStruct((B, 1, D), jnp.float32),
        scratch_shapes=[
            pltpu.VMEM((1, D), jnp.float32),
            pltpu.SMEM((1,), jnp.float32),
            pltpu.SMEM((1,), jnp.float32),
        ],
    )(t_i, arow3, A_rel, K_past, k_t.reshape(B, 1, D))
    return out.reshape(B, D)
